# baseline (speedup 1.0000x reference)
import functools

import jax
import jax.numpy as jnp
from jax import lax
from jax.experimental import pallas as pl
from jax.experimental.pallas import tpu as pltpu
from jax.experimental.pallas import tpu_sc as plsc

_H, _W = 20, 20
_VOCAB = 8
_B, _N, _C = 64, 32, 16
_ROWS = _B * _N * _C
_OUTROWS = _B * _N
_NW = 32
_L = 16

_mesh = plsc.VectorSubcoreMesh(core_axis_name="c", subcore_axis_name="s")
_cparams = pltpu.CompilerParams(needs_layout_passes=False,
                                use_tc_tiling_on_sc=True)


_NRB = 16
_RB = _OUTROWS // _NRB
_SLABS = _H * _W
_SPH = _SLABS // 2
_VPS = _C * _RB // _L
_K = 4


_BPRB = _RB // _N


def _fused_body(x_hbm, table_hbm, out_hbm, table_v, x_v, idxs_v, idxoff_v,
                buf0, buf1, osem0, osem1, tsem):
    wid = lax.axis_index("s") * 2 + lax.axis_index("c")
    rblk = wid % _NRB
    s0 = (wid // _NRB) * _SPH
    lanes = lax.iota(jnp.int32, _L)
    zeros = jnp.zeros((_L,), jnp.int32)

    pltpu.async_copy(table_hbm, table_v, tsem)

    b0 = rblk * _BPRB
    for bb in range(_BPRB):
        pltpu.sync_copy(x_hbm.at[b0 + bb], x_v)
        for n in range(_N):
            nv = jnp.full((_L,), n, jnp.int32)
            best = plsc.load_gather(x_v, [nv, lanes, zeros])
            bi = zeros
            for j in range(1, _VOCAB):
                v = plsc.load_gather(x_v, [nv, lanes,
                                           jnp.full((_L,), j, jnp.int32)])
                m = v > best
                best = jnp.where(m, v, best)
                bi = jnp.where(m, jnp.full((_L,), j, jnp.int32), bi)
            idxs_v[pl.ds((bb * _N + n) * _C, _C)] = bi

    for ct in range(2):
        for cc in range(8):
            c = ct * 8 + cc
            for rv in range(_RB // _L):
                g = plsc.load_gather(
                    idxs_v, [(jnp.full((_L,), rv * _L, jnp.int32) + lanes)
                             * _C + c])
                vpos = (ct * 8 + cc) * (_RB // _L) + rv
                idxoff_v[pl.ds(vpos * _L, _L)] = g * (_H * _W)

    pltpu.make_async_copy(table_hbm, table_v, tsem).wait()

    def out_at(s):
        return out_hbm.at[s // _W, pl.ds(s % _W, _K), slice(None),
                          pl.ds(rblk * _RB, _RB)]

    def do_group(k, buf, osem, drain):
        s = s0 + k
        if drain:
            pltpu.make_async_copy(buf, out_at(s - 2 * _K), osem).wait()
        svs = [jnp.full((_L,), s, jnp.int32) + j for j in range(_K)]
        for vp in range(0, _VPS, 2):
            ix = [idxoff_v[pl.ds((vp + g) * _L, _L)] for g in range(2)]
            vals = [plsc.load_gather(table_v, [ix[g] + svs[j]])
                    for g in range(2) for j in range(_K)]
            for g in range(2):
                c, rv = (vp + g) // (_RB // _L), (vp + g) % (_RB // _L)
                for j in range(_K):
                    buf[j, c, pl.ds(rv * _L, _L)] = vals[g * _K + j]
        pltpu.async_copy(buf, out_at(s), osem)

    do_group(0, buf0, osem0, False)
    do_group(_K, buf1, osem1, False)

    def pair(k):
        do_group(k, buf0, osem0, True)
        do_group(k + _K, buf1, osem1, True)

    pl.loop(2 * _K, _SPH, step=2 * _K)(pair)
    pltpu.make_async_copy(buf0, out_at(s0 + _SPH - 2 * _K), osem0).wait()
    pltpu.make_async_copy(buf1, out_at(s0 + _SPH - _K), osem1).wait()


@jax.jit
def _encode(x, table3):
    out_t = pl.kernel(
        _fused_body,
        out_type=jax.ShapeDtypeStruct((_H, _W, _C, _OUTROWS), jnp.float32),
        mesh=_mesh,
        compiler_params=_cparams,
        scratch_types=[
            pltpu.VMEM((_VOCAB * _H * _W,), jnp.float32),
            pltpu.VMEM((_N, _C, _VOCAB), jnp.float32),
            pltpu.VMEM((_RB * _C,), jnp.int32),
            pltpu.VMEM((_RB * _C,), jnp.int32),
            pltpu.VMEM((_K, _C, _RB), jnp.float32),
            pltpu.VMEM((_K, _C, _RB), jnp.float32),
            pltpu.SemaphoreType.DMA,
            pltpu.SemaphoreType.DMA,
            pltpu.SemaphoreType.DMA,
        ],
    )(x, table3)
    return out_t


def kernel(x, embed):
    out_t = _encode(x, embed.reshape(_VOCAB * _H * _W))
    return jnp.transpose(out_t, (3, 2, 0, 1))


# speedup vs baseline: 1.2800x; 1.2800x over previous
import functools

import jax
import jax.numpy as jnp
from jax import lax
from jax.experimental import pallas as pl
from jax.experimental.pallas import tpu as pltpu
from jax.experimental.pallas import tpu_sc as plsc

_H, _W = 20, 20
_VOCAB = 8
_B, _N, _C = 64, 32, 16
_ROWS = _B * _N * _C
_OUTROWS = _B * _N
_NW = 32
_L = 16

_BPW = _B // _NW
_NHALF = 16

_ORPW = _OUTROWS // _NW

_mesh = plsc.VectorSubcoreMesh(core_axis_name="c", subcore_axis_name="s")
_cparams = pltpu.CompilerParams(needs_layout_passes=False,
                                use_tc_tiling_on_sc=True)


def _argmax_body(x_hbm, idx_hbm, x_v, idx_v):
    wid = lax.axis_index("s") * 2 + lax.axis_index("c")
    b0 = wid * _BPW
    lanes = lax.iota(jnp.int32, _L)
    zeros = jnp.zeros((_L,), jnp.int32)

    for chunk in range(_BPW * 2):
        b = b0 + chunk // 2
        n0 = (chunk % 2) * _NHALF
        pltpu.sync_copy(x_hbm.at[b, pl.ds(n0, _NHALF)], x_v)
        for n in range(_NHALF):
            best = plsc.load_gather(x_v, [jnp.full((_L,), n, jnp.int32),
                                          lanes, zeros])
            bi = zeros
            for j in range(1, _VOCAB):
                v = plsc.load_gather(x_v, [jnp.full((_L,), n, jnp.int32),
                                           lanes,
                                           jnp.full((_L,), j, jnp.int32)])
                m = v > best
                best = jnp.where(m, v, best)
                bi = jnp.where(m, jnp.full((_L,), j, jnp.int32), bi)
            idx_v[pl.ds((chunk * _NHALF + n) * _C, _C)] = bi
    pltpu.sync_copy(idx_v, idx_hbm.at[pl.ds(wid * (_BPW * _N * _C),
                                            _BPW * _N * _C)])


_NRB = 16
_RB = _OUTROWS // _NRB
_SLABS = _H * _W
_SPH = _SLABS // 2
_VPS = _C * _RB // _L
_K = 4


def _lookup_body(table_hbm, idx_hbm, out_hbm, table_v, tablet_v, idxs_v,
                 idxoff_v, buf0, buf1, osem0, osem1):
    wid = lax.axis_index("s") * 2 + lax.axis_index("c")
    rblk = wid % _NRB
    s0 = (wid // _NRB) * _SPH
    lanes = lax.iota(jnp.int32, _L)

    pltpu.sync_copy(table_hbm, table_v)
    pltpu.sync_copy(idx_hbm.at[pl.ds(rblk * _RB * _C, _RB * _C)], idxs_v)

    pat = (lanes % _VOCAB) * _SLABS + lanes // _VOCAB

    @pl.loop(0, _SLABS * _VOCAB // _L)
    def _transpose(k):
        gv = plsc.load_gather(table_v, [pat + 2 * k])
        tablet_v[pl.ds(k * _L, _L)] = gv

    for ct in range(2):
        for cc in range(8):
            c = ct * 8 + cc
            for rv in range(_RB // _L):
                g = plsc.load_gather(
                    idxs_v, [(jnp.full((_L,), rv * _L, jnp.int32) + lanes)
                             * _C + c])
                vpos = (ct * 8 + cc) * (_RB // _L) + rv
                idxoff_v[pl.ds(vpos * _L, _L)] = g

    bufs = (buf0, buf1)
    osems = (osem0, osem1)

    def out_at(s):
        return out_hbm.at[s // _W, pl.ds(s % _W, _K), slice(None),
                          pl.ds(rblk * _RB, _RB)]

    def do_group(k, buf, osem, drain):
        s = s0 + k
        if drain:
            pltpu.make_async_copy(buf, out_at(s - 2 * _K), osem).wait()
        tviews = [tablet_v.at[pl.ds((s + j) * _VOCAB, _VOCAB)]
                  for j in range(_K)]
        for vp in range(0, _VPS, 2):
            ix = [idxoff_v[pl.ds((vp + g) * _L, _L)] for g in range(2)]
            vals = [plsc.load_gather(tviews[j], [ix[g]])
                    for g in range(2) for j in range(_K)]
            for g in range(2):
                c, rv = (vp + g) // (_RB // _L), (vp + g) % (_RB // _L)
                for j in range(_K):
                    buf[j, c, pl.ds(rv * _L, _L)] = vals[g * _K + j]
        pltpu.async_copy(buf, out_at(s), osem)

    do_group(0, buf0, osem0, False)
    do_group(_K, buf1, osem1, False)

    def pair(k):
        do_group(k, buf0, osem0, True)
        do_group(k + _K, buf1, osem1, True)

    pl.loop(2 * _K, _SPH, step=2 * _K)(pair)
    pltpu.make_async_copy(buf0, out_at(s0 + _SPH - 2 * _K), osem0).wait()
    pltpu.make_async_copy(buf1, out_at(s0 + _SPH - _K), osem1).wait()


@jax.jit
def _encode(x, table3):
    idx = pl.kernel(
        _argmax_body,
        out_type=jax.ShapeDtypeStruct((_ROWS,), jnp.int32),
        mesh=_mesh,
        compiler_params=_cparams,
        scratch_types=[
            pltpu.VMEM((_NHALF, _C, _VOCAB), jnp.float32),
            pltpu.VMEM((_BPW * _N * _C,), jnp.int32),
        ],
    )(x)
    out_t = pl.kernel(
        _lookup_body,
        out_type=jax.ShapeDtypeStruct((_H, _W, _C, _OUTROWS), jnp.float32),
        mesh=_mesh,
        compiler_params=_cparams,
        scratch_types=[
            pltpu.VMEM((_VOCAB * _H * _W,), jnp.float32),
            pltpu.VMEM((_VOCAB * _H * _W,), jnp.float32),
            pltpu.VMEM((_RB * _C,), jnp.int32),
            pltpu.VMEM((_RB * _C,), jnp.int32),
            pltpu.VMEM((_K, _C, _RB), jnp.float32),
            pltpu.VMEM((_K, _C, _RB), jnp.float32),
            pltpu.SemaphoreType.DMA,
            pltpu.SemaphoreType.DMA,
        ],
    )(table3, idx)
    return out_t


def kernel(x, embed):
    out_t = _encode(x, embed.reshape(_VOCAB * _H * _W))
    return jnp.transpose(out_t, (3, 2, 0, 1))


# speedup vs baseline: 1.3450x; 1.0508x over previous
import functools

import jax
import jax.numpy as jnp
from jax import lax
from jax.experimental import pallas as pl
from jax.experimental.pallas import tpu as pltpu
from jax.experimental.pallas import tpu_sc as plsc

_H, _W = 20, 20
_VOCAB = 8
_B, _N, _C = 64, 32, 16
_ROWS = _B * _N * _C
_OUTROWS = _B * _N
_NW = 32
_L = 16

_BPW = _B // _NW
_NHALF = 16

_ORPW = _OUTROWS // _NW

_mesh = plsc.VectorSubcoreMesh(core_axis_name="c", subcore_axis_name="s")
_cparams = pltpu.CompilerParams(needs_layout_passes=False,
                                use_tc_tiling_on_sc=True)


def _argmax_body(x_hbm, idx_hbm, x_v0, x_v1, idx_v, xsem0, xsem1):
    wid = lax.axis_index("s") * 2 + lax.axis_index("c")
    b0 = wid * _BPW
    lanes = lax.iota(jnp.int32, _L)
    zeros = jnp.zeros((_L,), jnp.int32)

    def chunk_src(chunk):
        return x_hbm.at[b0 + chunk // 2, pl.ds((chunk % 2) * _NHALF,
                                               _NHALF)]

    xbufs = (x_v0, x_v1)
    xsems = (xsem0, xsem1)
    pltpu.async_copy(chunk_src(0), x_v0, xsem0)
    pltpu.async_copy(chunk_src(1), x_v1, xsem1)

    for chunk in range(_BPW * 2):
        x_v, xsem = xbufs[chunk % 2], xsems[chunk % 2]
        pltpu.make_async_copy(chunk_src(chunk), x_v, xsem).wait()
        for n in range(_NHALF):
            best = plsc.load_gather(x_v, [jnp.full((_L,), n, jnp.int32),
                                          lanes, zeros])
            bi = zeros
            for j in range(1, _VOCAB):
                v = plsc.load_gather(x_v, [jnp.full((_L,), n, jnp.int32),
                                           lanes,
                                           jnp.full((_L,), j, jnp.int32)])
                m = v > best
                best = jnp.where(m, v, best)
                bi = jnp.where(m, jnp.full((_L,), j, jnp.int32), bi)
            idx_v[pl.ds((chunk * _NHALF + n) * _C, _C)] = bi
        if chunk + 2 < _BPW * 2:
            pltpu.async_copy(chunk_src(chunk + 2), x_v, xsem)
    pltpu.sync_copy(idx_v, idx_hbm.at[pl.ds(wid * (_BPW * _N * _C),
                                            _BPW * _N * _C)])


_NRB = 16
_RB = _OUTROWS // _NRB
_SLABS = _H * _W
_SPH = _SLABS // 2
_VPS = _C * _RB // _L
_K = 4


def _lookup_body(table_hbm, idx_hbm, out_hbm, table_v, tablet_v, idxs_v,
                 idxoff_v, buf0, buf1, osem0, osem1, isem):
    wid = lax.axis_index("s") * 2 + lax.axis_index("c")
    rblk = wid % _NRB
    s0 = (wid // _NRB) * _SPH
    lanes = lax.iota(jnp.int32, _L)

    idx_src = idx_hbm.at[pl.ds(rblk * _RB * _C, _RB * _C)]
    pltpu.async_copy(idx_src, idxs_v, isem)
    pltpu.sync_copy(table_hbm, table_v)

    pat = (lanes % _VOCAB) * _SLABS + lanes // _VOCAB

    @pl.loop(0, _SLABS * _VOCAB // _L)
    def _transpose(k):
        gv = plsc.load_gather(table_v, [pat + 2 * k])
        tablet_v[pl.ds(k * _L, _L)] = gv

    pltpu.make_async_copy(idx_src, idxs_v, isem).wait()

    for ct in range(2):
        for cc in range(8):
            c = ct * 8 + cc
            for rv in range(_RB // _L):
                g = plsc.load_gather(
                    idxs_v, [(jnp.full((_L,), rv * _L, jnp.int32) + lanes)
                             * _C + c])
                vpos = (ct * 8 + cc) * (_RB // _L) + rv
                idxoff_v[pl.ds(vpos * _L, _L)] = g

    bufs = (buf0, buf1)
    osems = (osem0, osem1)

    def out_at(s):
        return out_hbm.at[s // _W, pl.ds(s % _W, _K), slice(None),
                          pl.ds(rblk * _RB, _RB)]

    def do_group(k, buf, osem, drain):
        s = s0 + k
        if drain:
            pltpu.make_async_copy(buf, out_at(s - 2 * _K), osem).wait()
        tviews = [tablet_v.at[pl.ds((s + j) * _VOCAB, _VOCAB)]
                  for j in range(_K)]
        for vp in range(0, _VPS, 2):
            ix = [idxoff_v[pl.ds((vp + g) * _L, _L)] for g in range(2)]
            vals = [plsc.load_gather(tviews[j], [ix[g]])
                    for g in range(2) for j in range(_K)]
            for g in range(2):
                c, rv = (vp + g) // (_RB // _L), (vp + g) % (_RB // _L)
                for j in range(_K):
                    buf[j, c, pl.ds(rv * _L, _L)] = vals[g * _K + j]
        pltpu.async_copy(buf, out_at(s), osem)

    do_group(0, buf0, osem0, False)
    do_group(_K, buf1, osem1, False)

    def pair(k):
        do_group(k, buf0, osem0, True)
        do_group(k + _K, buf1, osem1, True)

    pl.loop(2 * _K, _SPH, step=2 * _K)(pair)
    pltpu.make_async_copy(buf0, out_at(s0 + _SPH - 2 * _K), osem0).wait()
    pltpu.make_async_copy(buf1, out_at(s0 + _SPH - _K), osem1).wait()


@jax.jit
def _encode(x, table3):
    idx = pl.kernel(
        _argmax_body,
        out_type=jax.ShapeDtypeStruct((_ROWS,), jnp.int32),
        mesh=_mesh,
        compiler_params=_cparams,
        scratch_types=[
            pltpu.VMEM((_NHALF, _C, _VOCAB), jnp.float32),
            pltpu.VMEM((_NHALF, _C, _VOCAB), jnp.float32),
            pltpu.VMEM((_BPW * _N * _C,), jnp.int32),
            pltpu.SemaphoreType.DMA,
            pltpu.SemaphoreType.DMA,
        ],
    )(x)
    out_t = pl.kernel(
        _lookup_body,
        out_type=jax.ShapeDtypeStruct((_H, _W, _C, _OUTROWS), jnp.float32),
        mesh=_mesh,
        compiler_params=_cparams,
        scratch_types=[
            pltpu.VMEM((_VOCAB * _H * _W,), jnp.float32),
            pltpu.VMEM((_VOCAB * _H * _W,), jnp.float32),
            pltpu.VMEM((_RB * _C,), jnp.int32),
            pltpu.VMEM((_RB * _C,), jnp.int32),
            pltpu.VMEM((_K, _C, _RB), jnp.float32),
            pltpu.VMEM((_K, _C, _RB), jnp.float32),
            pltpu.SemaphoreType.DMA,
            pltpu.SemaphoreType.DMA,
            pltpu.SemaphoreType.DMA,
        ],
    )(table3, idx)
    return out_t


def kernel(x, embed):
    out_t = _encode(x, embed.reshape(_VOCAB * _H * _W))
    return jnp.transpose(out_t, (3, 2, 0, 1))


# speedup vs baseline: 1.3579x; 1.0096x over previous
import functools

import jax
import jax.numpy as jnp
from jax import lax
from jax.experimental import pallas as pl
from jax.experimental.pallas import tpu as pltpu
from jax.experimental.pallas import tpu_sc as plsc

_H, _W = 20, 20
_VOCAB = 8
_B, _N, _C = 64, 32, 16
_ROWS = _B * _N * _C
_OUTROWS = _B * _N
_NW = 32
_L = 16

_BPW = _B // _NW
_NHALF = 16

_ORPW = _OUTROWS // _NW

_mesh = plsc.VectorSubcoreMesh(core_axis_name="c", subcore_axis_name="s")
_cparams = pltpu.CompilerParams(needs_layout_passes=False,
                                use_tc_tiling_on_sc=True)


_NCH = 8
_NQ = _N // (_NCH // _BPW)


def _argmax_body(x_hbm, idx_hbm, x_v0, x_v1, x_v2, x_v3, idx_v,
                 xsem0, xsem1, xsem2, xsem3):
    wid = lax.axis_index("s") * 2 + lax.axis_index("c")
    b0 = wid * _BPW
    lanes = lax.iota(jnp.int32, _L)
    zeros = jnp.zeros((_L,), jnp.int32)

    def chunk_src(chunk):
        return x_hbm.at[b0 + chunk // 4, pl.ds((chunk % 4) * _NQ, _NQ)]

    xbufs = (x_v0, x_v1, x_v2, x_v3)
    xsems = (xsem0, xsem1, xsem2, xsem3)
    for i in range(4):
        pltpu.async_copy(chunk_src(i), xbufs[i], xsems[i])

    for chunk in range(_NCH):
        x_v, xsem = xbufs[chunk % 4], xsems[chunk % 4]
        pltpu.make_async_copy(chunk_src(chunk), x_v, xsem).wait()
        for n in range(_NQ):
            best = plsc.load_gather(x_v, [jnp.full((_L,), n, jnp.int32),
                                          lanes, zeros])
            bi = zeros
            for j in range(1, _VOCAB):
                v = plsc.load_gather(x_v, [jnp.full((_L,), n, jnp.int32),
                                           lanes,
                                           jnp.full((_L,), j, jnp.int32)])
                m = v > best
                best = jnp.where(m, v, best)
                bi = jnp.where(m, jnp.full((_L,), j, jnp.int32), bi)
            idx_v[pl.ds((chunk * _NQ + n) * _C, _C)] = bi
        if chunk + 4 < _NCH:
            pltpu.async_copy(chunk_src(chunk + 4), x_v, xsem)
    pltpu.sync_copy(idx_v, idx_hbm.at[pl.ds(wid * (_BPW * _N * _C),
                                            _BPW * _N * _C)])


_NRB = 16
_RB = _OUTROWS // _NRB
_SLABS = _H * _W
_SPH = _SLABS // 2
_VPS = _C * _RB // _L
_K = 4


def _lookup_body(table_hbm, idx_hbm, out_hbm, table_v, tablet_v, idxs_v,
                 idxoff_v, buf0, buf1, osem0, osem1, isem):
    wid = lax.axis_index("s") * 2 + lax.axis_index("c")
    rblk = wid % _NRB
    s0 = (wid // _NRB) * _SPH
    lanes = lax.iota(jnp.int32, _L)

    idx_src = idx_hbm.at[pl.ds(rblk * _RB * _C, _RB * _C)]
    pltpu.async_copy(idx_src, idxs_v, isem)
    pltpu.sync_copy(table_hbm, table_v)

    pat = (lanes % _VOCAB) * _SLABS + lanes // _VOCAB

    @pl.loop(0, _SLABS * _VOCAB // _L)
    def _transpose(k):
        gv = plsc.load_gather(table_v, [pat + 2 * k])
        tablet_v[pl.ds(k * _L, _L)] = gv

    pltpu.make_async_copy(idx_src, idxs_v, isem).wait()

    for ct in range(2):
        for cc in range(8):
            c = ct * 8 + cc
            for rv in range(_RB // _L):
                g = plsc.load_gather(
                    idxs_v, [(jnp.full((_L,), rv * _L, jnp.int32) + lanes)
                             * _C + c])
                vpos = (ct * 8 + cc) * (_RB // _L) + rv
                idxoff_v[pl.ds(vpos * _L, _L)] = g

    bufs = (buf0, buf1)
    osems = (osem0, osem1)

    def out_at(s):
        return out_hbm.at[s // _W, pl.ds(s % _W, _K), slice(None),
                          pl.ds(rblk * _RB, _RB)]

    def do_group(k, buf, osem, drain):
        s = s0 + k
        if drain:
            pltpu.make_async_copy(buf, out_at(s - 2 * _K), osem).wait()
        tviews = [tablet_v.at[pl.ds((s + j) * _VOCAB, _VOCAB)]
                  for j in range(_K)]
        for vp in range(0, _VPS, 2):
            ix = [idxoff_v[pl.ds((vp + g) * _L, _L)] for g in range(2)]
            vals = [plsc.load_gather(tviews[j], [ix[g]])
                    for g in range(2) for j in range(_K)]
            for g in range(2):
                c, rv = (vp + g) // (_RB // _L), (vp + g) % (_RB // _L)
                for j in range(_K):
                    buf[j, c, pl.ds(rv * _L, _L)] = vals[g * _K + j]
        pltpu.async_copy(buf, out_at(s), osem)

    do_group(0, buf0, osem0, False)
    do_group(_K, buf1, osem1, False)

    def pair(k):
        do_group(k, buf0, osem0, True)
        do_group(k + _K, buf1, osem1, True)

    pl.loop(2 * _K, _SPH, step=2 * _K)(pair)
    pltpu.make_async_copy(buf0, out_at(s0 + _SPH - 2 * _K), osem0).wait()
    pltpu.make_async_copy(buf1, out_at(s0 + _SPH - _K), osem1).wait()


@jax.jit
def _encode(x, table3):
    idx = pl.kernel(
        _argmax_body,
        out_type=jax.ShapeDtypeStruct((_ROWS,), jnp.int32),
        mesh=_mesh,
        compiler_params=_cparams,
        scratch_types=[
            pltpu.VMEM((_NQ, _C, _VOCAB), jnp.float32),
            pltpu.VMEM((_NQ, _C, _VOCAB), jnp.float32),
            pltpu.VMEM((_NQ, _C, _VOCAB), jnp.float32),
            pltpu.VMEM((_NQ, _C, _VOCAB), jnp.float32),
            pltpu.VMEM((_BPW * _N * _C,), jnp.int32),
            pltpu.SemaphoreType.DMA,
            pltpu.SemaphoreType.DMA,
            pltpu.SemaphoreType.DMA,
            pltpu.SemaphoreType.DMA,
        ],
    )(x)
    out_t = pl.kernel(
        _lookup_body,
        out_type=jax.ShapeDtypeStruct((_H, _W, _C, _OUTROWS), jnp.float32),
        mesh=_mesh,
        compiler_params=_cparams,
        scratch_types=[
            pltpu.VMEM((_VOCAB * _H * _W,), jnp.float32),
            pltpu.VMEM((_VOCAB * _H * _W,), jnp.float32),
            pltpu.VMEM((_RB * _C,), jnp.int32),
            pltpu.VMEM((_RB * _C,), jnp.int32),
            pltpu.VMEM((_K, _C, _RB), jnp.float32),
            pltpu.VMEM((_K, _C, _RB), jnp.float32),
            pltpu.SemaphoreType.DMA,
            pltpu.SemaphoreType.DMA,
            pltpu.SemaphoreType.DMA,
        ],
    )(table3, idx)
    return out_t


def kernel(x, embed):
    out_t = _encode(x, embed.reshape(_VOCAB * _H * _W))
    return jnp.transpose(out_t, (3, 2, 0, 1))
